# Initial kernel scaffold; baseline (speedup 1.0000x reference)
import jax
import jax.numpy as jnp
from jax.experimental import pallas as pl


def kernel(x, W_enc, b_enc, enc_bias, W_dec):
    raise NotImplementedError("write your pallas kernel here")


# speedup vs baseline: 19.4585x; 19.4585x over previous
import jax
import jax.numpy as jnp
from jax.experimental import pallas as pl
from jax.experimental.pallas import tpu as pltpu

D_MODEL_C = 1024
D_LAT_C = 4096
K_C = 32
ROWS = 2048
BLK = 256


def _body(x_ref, we_ref, b1_ref, b2_ref, wd_ref, lat_ref, rec_ref):
    x = x_ref[...]
    pre = jax.lax.dot_general(
        x, we_ref[...], (((1,), (1,)), ((), ())),
        preferred_element_type=jnp.float32)
    pre = pre + b1_ref[...] + b2_ref[...]

    bits = jax.lax.bitcast_convert_type(pre, jnp.uint32)
    neg = bits >= jnp.uint32(0x80000000)
    key = jnp.where(neg, ~bits, bits | jnp.uint32(0x80000000))

    def step(b, prefix):
        bit = jax.lax.shift_left(jnp.uint32(1), jnp.uint32(31) - b.astype(jnp.uint32))
        cand = prefix | bit
        cnt = jnp.sum((key >= cand).astype(jnp.int32), axis=1, keepdims=True)
        return jnp.where(cnt >= K_C, cand, prefix)

    prefix = jax.lax.fori_loop(
        0, 32, step, jnp.zeros((BLK, 1), jnp.uint32), unroll=True)

    lat = jnp.where(key >= prefix, jnp.maximum(pre, 0.0), 0.0)
    lat_ref[...] = lat
    rec_ref[...] = jax.lax.dot_general(
        lat, wd_ref[...], (((1,), (1,)), ((), ())),
        preferred_element_type=jnp.float32)


def kernel(x, W_enc, b_enc, enc_bias, W_dec):
    B, S, D = x.shape
    x2 = x.reshape(B * S, D)
    b1 = b_enc.reshape(1, D_LAT_C)
    b2 = enc_bias.reshape(1, D_LAT_C)
    grid = (B * S) // BLK

    lat2, rec2 = pl.pallas_call(
        _body,
        grid=(grid,),
        in_specs=[
            pl.BlockSpec((BLK, D), lambda i: (i, 0)),
            pl.BlockSpec((D_LAT_C, D), lambda i: (0, 0)),
            pl.BlockSpec((1, D_LAT_C), lambda i: (0, 0)),
            pl.BlockSpec((1, D_LAT_C), lambda i: (0, 0)),
            pl.BlockSpec((D, D_LAT_C), lambda i: (0, 0)),
        ],
        out_specs=[
            pl.BlockSpec((BLK, D_LAT_C), lambda i: (i, 0)),
            pl.BlockSpec((BLK, D), lambda i: (i, 0)),
        ],
        out_shape=[
            jax.ShapeDtypeStruct((B * S, D_LAT_C), jnp.float32),
            jax.ShapeDtypeStruct((B * S, D), jnp.float32),
        ],
        compiler_params=pltpu.CompilerParams(
            dimension_semantics=("arbitrary",),
        ),
    )(x2, W_enc, b1, b2, W_dec)

    return rec2.reshape(B, S, D), lat2.reshape(B, S, D_LAT_C)


# speedup vs baseline: 19.5469x; 1.0045x over previous
import jax
import jax.numpy as jnp
from jax.experimental import pallas as pl
from jax.experimental.pallas import tpu as pltpu

D_MODEL_C = 1024
D_LAT_C = 4096
K_C = 32
ROWS = 2048
BLK = 256


def _body(x_ref, we_ref, b1_ref, b2_ref, wd_ref, lat_ref, rec_ref):
    x = x_ref[...]
    pre = jax.lax.dot_general(
        x, we_ref[...], (((1,), (1,)), ((), ())),
        preferred_element_type=jnp.float32)
    pre = pre + b1_ref[...] + b2_ref[...]

    bits = jax.lax.bitcast_convert_type(pre, jnp.uint32)
    neg = bits >= jnp.uint32(0x80000000)
    key = jnp.where(neg, ~bits, bits | jnp.uint32(0x80000000))

    def step(b, prefix):
        bit = jax.lax.shift_left(jnp.uint32(1), jnp.uint32(31) - b.astype(jnp.uint32))
        cand = prefix | bit
        cnt = jnp.sum((key >= cand).astype(jnp.int32), axis=1, keepdims=True)
        return jnp.where(cnt >= K_C, cand, prefix)

    prefix = jax.lax.fori_loop(
        0, 32, step, jnp.zeros((BLK, 1), jnp.uint32), unroll=True)

    lat = jnp.where(key >= prefix, jnp.maximum(pre, 0.0), 0.0)
    lat_ref[...] = lat
    rec_ref[...] = jax.lax.dot_general(
        lat.astype(jnp.bfloat16), wd_ref[...].astype(jnp.bfloat16),
        (((1,), (1,)), ((), ())),
        preferred_element_type=jnp.float32)


def kernel(x, W_enc, b_enc, enc_bias, W_dec):
    B, S, D = x.shape
    x2 = x.reshape(B * S, D)
    b1 = b_enc.reshape(1, D_LAT_C)
    b2 = enc_bias.reshape(1, D_LAT_C)
    grid = (B * S) // BLK

    lat2, rec2 = pl.pallas_call(
        _body,
        grid=(grid,),
        in_specs=[
            pl.BlockSpec((BLK, D), lambda i: (i, 0)),
            pl.BlockSpec((D_LAT_C, D), lambda i: (0, 0)),
            pl.BlockSpec((1, D_LAT_C), lambda i: (0, 0)),
            pl.BlockSpec((1, D_LAT_C), lambda i: (0, 0)),
            pl.BlockSpec((D, D_LAT_C), lambda i: (0, 0)),
        ],
        out_specs=[
            pl.BlockSpec((BLK, D_LAT_C), lambda i: (i, 0)),
            pl.BlockSpec((BLK, D), lambda i: (i, 0)),
        ],
        out_shape=[
            jax.ShapeDtypeStruct((B * S, D_LAT_C), jnp.float32),
            jax.ShapeDtypeStruct((B * S, D), jnp.float32),
        ],
        compiler_params=pltpu.CompilerParams(
            dimension_semantics=("arbitrary",),
        ),
    )(x2, W_enc, b1, b2, W_dec)

    return rec2.reshape(B, S, D), lat2.reshape(B, S, D_LAT_C)
